# Initial kernel scaffold; baseline (speedup 1.0000x reference)
import jax
import jax.numpy as jnp
from jax.experimental import pallas as pl


def kernel(X, A, alpha, beta, X0, W, b):
    raise NotImplementedError("write your pallas kernel here")


# speedup vs baseline: 1.0059x; 1.0059x over previous
import jax
import jax.numpy as jnp
from jax.experimental import pallas as pl
from jax.experimental.pallas import tpu as pltpu


def _prep_kernel(beta_ref, x0_ref, w_ref, b_ref, y_ref, c_ref):
    beta = beta_ref[0, 0]
    x0 = x0_ref[:]
    x0wt = jax.lax.dot_general(
        x0, w_ref[:], dimension_numbers=(((1,), (1,)), ((), ())),
        preferred_element_type=jnp.float32)
    y_ref[:] = (1.0 - beta) * x0 + beta * x0wt
    c_ref[:] = beta * b_ref[:]


def _main_kernel(alpha_ref, a_ref, y_ref, c_ref, o_ref, *, bm):
    i = pl.program_id(0)
    alpha = alpha_ref[0, 0]
    acc = jnp.dot(a_ref[:], y_ref[:], preferred_element_type=jnp.float32)
    yi = y_ref[pl.ds(i * bm, bm), :]
    o_ref[:] = (1.0 - alpha) * acc + alpha * yi + c_ref[:]


def kernel(X, A, alpha, beta, X0, W, b):
    del X
    n, k = A.shape
    d = X0.shape[1]

    beta2 = jnp.reshape(beta.astype(jnp.float32), (1, 1))
    alpha2 = jnp.reshape(alpha.astype(jnp.float32), (1, 1))
    b2 = jnp.reshape(b, (1, d))

    y, c = pl.pallas_call(
        _prep_kernel,
        in_specs=[
            pl.BlockSpec(memory_space=pltpu.SMEM),
            pl.BlockSpec((k, d), lambda: (0, 0)),
            pl.BlockSpec((d, d), lambda: (0, 0)),
            pl.BlockSpec((1, d), lambda: (0, 0)),
        ],
        out_specs=[
            pl.BlockSpec((k, d), lambda: (0, 0)),
            pl.BlockSpec((1, d), lambda: (0, 0)),
        ],
        out_shape=[
            jax.ShapeDtypeStruct((k, d), jnp.float32),
            jax.ShapeDtypeStruct((1, d), jnp.float32),
        ],
    )(beta2, X0, W, b2)

    bm = 200
    grid = (n // bm,)

    import functools
    out = pl.pallas_call(
        functools.partial(_main_kernel, bm=bm),
        grid=grid,
        in_specs=[
            pl.BlockSpec(memory_space=pltpu.SMEM),
            pl.BlockSpec((bm, k), lambda i: (i, 0)),
            pl.BlockSpec((k, d), lambda i: (0, 0)),
            pl.BlockSpec((1, d), lambda i: (0, 0)),
        ],
        out_specs=pl.BlockSpec((bm, d), lambda i: (i, 0)),
        out_shape=jax.ShapeDtypeStruct((n, d), jnp.float32),
        compiler_params=pltpu.CompilerParams(
            dimension_semantics=("parallel",),
        ),
    )(alpha2, A, y, c)
    return out


# speedup vs baseline: 1.0217x; 1.0157x over previous
import functools

import jax
import jax.numpy as jnp
from jax.experimental import pallas as pl
from jax.experimental.pallas import tpu as pltpu


def _fused_kernel(alpha_ref, beta_ref, a_ref, x0_ref, w_ref, b_ref, o_ref,
                  *, bm):
    i = pl.program_id(0)
    alpha = alpha_ref[0, 0]
    beta = beta_ref[0, 0]
    acc = jnp.dot(a_ref[:], x0_ref[:], preferred_element_type=jnp.float32)
    xi = (1.0 - alpha) * acc + alpha * x0_ref[pl.ds(i * bm, bm), :]
    xiwt = jax.lax.dot_general(
        xi, w_ref[:], dimension_numbers=(((1,), (1,)), ((), ())),
        preferred_element_type=jnp.float32)
    o_ref[:] = (1.0 - beta) * xi + beta * (xiwt + b_ref[:])


def kernel(X, A, alpha, beta, X0, W, b):
    del X
    n, k = A.shape
    d = X0.shape[1]

    alpha2 = jnp.reshape(alpha.astype(jnp.float32), (1, 1))
    beta2 = jnp.reshape(beta.astype(jnp.float32), (1, 1))
    b2 = jnp.reshape(b, (1, d))

    bm = 200
    grid = (n // bm,)

    out = pl.pallas_call(
        functools.partial(_fused_kernel, bm=bm),
        grid=grid,
        in_specs=[
            pl.BlockSpec(memory_space=pltpu.SMEM),
            pl.BlockSpec(memory_space=pltpu.SMEM),
            pl.BlockSpec((bm, k), lambda i: (i, 0)),
            pl.BlockSpec((k, d), lambda i: (0, 0)),
            pl.BlockSpec((d, d), lambda i: (0, 0)),
            pl.BlockSpec((1, d), lambda i: (0, 0)),
        ],
        out_specs=pl.BlockSpec((bm, d), lambda i: (i, 0)),
        out_shape=jax.ShapeDtypeStruct((n, d), jnp.float32),
        compiler_params=pltpu.CompilerParams(
            dimension_semantics=("parallel",),
        ),
    )(alpha2, beta2, A, X0, W, b2)
    return out


# speedup vs baseline: 1.0437x; 1.0215x over previous
import functools

import jax
import jax.numpy as jnp
from jax.experimental import pallas as pl
from jax.experimental.pallas import tpu as pltpu


def _fused_kernel(alpha_ref, beta_ref, a_ref, x0_ref, w_ref, b_ref, o_ref,
                  *, bm):
    i = pl.program_id(0)
    alpha = alpha_ref[0, 0]
    beta = beta_ref[0, 0]
    acc = jnp.dot(a_ref[:], x0_ref[:], preferred_element_type=jnp.float32)
    xi = (1.0 - alpha) * acc + alpha * x0_ref[pl.ds(i * bm, bm), :]
    xiwt = jax.lax.dot_general(
        xi, w_ref[:], dimension_numbers=(((1,), (1,)), ((), ())),
        preferred_element_type=jnp.float32)
    o_ref[:] = (1.0 - beta) * xi + beta * (xiwt + b_ref[:])


def kernel(X, A, alpha, beta, X0, W, b):
    del X
    n, k = A.shape
    d = X0.shape[1]

    alpha2 = jnp.reshape(alpha.astype(jnp.float32), (1, 1))
    beta2 = jnp.reshape(beta.astype(jnp.float32), (1, 1))
    b2 = jnp.reshape(b, (1, d))

    bm = 400
    grid = (n // bm,)

    out = pl.pallas_call(
        functools.partial(_fused_kernel, bm=bm),
        grid=grid,
        in_specs=[
            pl.BlockSpec(memory_space=pltpu.SMEM),
            pl.BlockSpec(memory_space=pltpu.SMEM),
            pl.BlockSpec((bm, k), lambda i: (i, 0)),
            pl.BlockSpec((k, d), lambda i: (0, 0)),
            pl.BlockSpec((d, d), lambda i: (0, 0)),
            pl.BlockSpec((1, d), lambda i: (0, 0)),
        ],
        out_specs=pl.BlockSpec((bm, d), lambda i: (i, 0)),
        out_shape=jax.ShapeDtypeStruct((n, d), jnp.float32),
        compiler_params=pltpu.CompilerParams(
            dimension_semantics=("parallel",),
        ),
    )(alpha2, beta2, A, X0, W, b2)
    return out
